# baseline (speedup 1.0000x reference)
import functools

import jax
import jax.numpy as jnp
from jax import lax
from jax.experimental import pallas as pl
from jax.experimental.pallas import tpu as pltpu
from jax.experimental.pallas import tpu_sc as plsc

_NC, _NS = 2, 16
_NW = _NC * _NS
_EMB = 32
_CHUNK = 128


def _sc_gather(u2, v2, n2, gmf_u, gmf_v, u_t, v_t, B, BN):
    cb = B // _NW
    cn = BN // _NW
    kb = cb // _CHUNK
    kn = cn // _CHUNK
    mesh = plsc.VectorSubcoreMesh(core_axis_name="c", subcore_axis_name="s",
                                  num_cores=_NC, num_subcores=_NS)
    out_type = tuple(
        jax.ShapeDtypeStruct((sz, _EMB), jnp.float32)
        for sz in (B, B, B, B, BN, BN)
    )
    scratch = [
        pltpu.VMEM((kb, _CHUNK), jnp.int32),
        pltpu.VMEM((kb, _CHUNK), jnp.int32),
        pltpu.VMEM((kn, _CHUNK), jnp.int32),
        pltpu.VMEM((cn, _EMB), jnp.float32),
        pltpu.SemaphoreType.DMA,
    ]

    @functools.partial(pl.kernel, mesh=mesh, out_type=out_type,
                       scratch_types=scratch,
                       compiler_params=pltpu.CompilerParams(
                           use_tc_tiling_on_sc=False))
    def k(u_h, v_h, n_h, gmfu_h, gmfv_h, ut_h, vt_h,
          gu_o, gv_o, ue_o, ve_o, gn_o, ne_o,
          ui_v, vi_v, ni_v, rows_v, sem):
        w = lax.axis_index("s") * _NC + lax.axis_index("c")
        pltpu.sync_copy(u_h.at[w], ui_v)
        pltpu.sync_copy(v_h.at[w], vi_v)
        pltpu.sync_copy(n_h.at[w], ni_v)

        def do_set(tbl, idx2, nchunks, out, base, cnt):
            descs = []
            for i in range(nchunks):
                descs.append(pltpu.async_copy(
                    tbl.at[idx2.at[i]],
                    rows_v.at[pl.ds(i * _CHUNK, _CHUNK)], sem))
            for d in descs:
                d.wait()
            pltpu.sync_copy(rows_v.at[pl.ds(0, cnt)], out.at[pl.ds(base, cnt)])

        bu = w * cb
        bn = w * cn
        do_set(gmfu_h, ui_v, kb, gu_o, bu, cb)
        do_set(gmfv_h, vi_v, kb, gv_o, bu, cb)
        do_set(ut_h, ui_v, kb, ue_o, bu, cb)
        do_set(vt_h, vi_v, kb, ve_o, bu, cb)
        do_set(gmfv_h, ni_v, kn, gn_o, bn, cn)
        do_set(vt_h, ni_v, kn, ne_o, bn, cn)

    return k(u2, v2, n2, gmf_u, gmf_v, u_t, v_t)


def _transpose_body(a_r, b_r, c_r, d_r, ao_r, bo_r, co_r, do_r):
    ao_r[...] = a_r[...].T
    bo_r[...] = b_r[...].T
    co_r[...] = c_r[...].T
    do_r[...] = d_r[...].T


def _tc_transpose4(a, b, c, d):
    n = a.shape[1]
    blk = 2048
    grid = pl.cdiv(n, blk)
    in_spec = pl.BlockSpec((_EMB, blk), lambda i: (0, i))
    out_spec = pl.BlockSpec((blk, _EMB), lambda i: (i, 0))
    return pl.pallas_call(
        _transpose_body,
        grid=(grid,),
        in_specs=[in_spec] * 4,
        out_specs=[out_spec] * 4,
        out_shape=[jax.ShapeDtypeStruct((n, _EMB), jnp.float32)] * 4,
    )(a, b, c, d)


def _tc_body(gu_r, gv_r, ue_r, ve_r, gn_r, ne_r,
             w1t_r, w1b_r, w2_r, wp1_r, wp2_r, b1_r, b2_r, bp_r,
             po_r, pn_r):
    w1b = w1b_r[...]
    w2 = w2_r[...]
    wp1 = wp1_r[...]
    wp2 = wp2_r[...]
    b1 = b1_r[...]
    b2 = b2_r[...]
    bp = bp_r[0, 0]
    gu = gu_r[...]
    au = jnp.dot(ue_r[...], w1t_r[...], preferred_element_type=jnp.float32)

    def head(a_u, other, gmf_prod):
        h1 = jnp.maximum(
            a_u + jnp.dot(other, w1b, preferred_element_type=jnp.float32) + b1,
            0.0)
        h2 = jnp.maximum(
            jnp.dot(h1, w2, preferred_element_type=jnp.float32) + b2, 0.0)
        return (jnp.sum(gmf_prod * wp1, axis=1)
                + jnp.sum(h2 * wp2, axis=1) + bp)

    po_r[:] = head(au, ve_r[...], gu * gv_r[...])
    for j in range(4):
        pn_r[j, :] = head(au, ne_r[j], gu * gn_r[j])


def _tc_dense(gu, gv, ue, ve, gn, ne, w1t, w1b, w2, wp1, wp2, b1, b2, bp):
    B = gu.shape[0]
    blk = 2048
    nb = B // blk
    row_spec = pl.BlockSpec((blk, _EMB), lambda i: (i, 0))
    neg_spec = pl.BlockSpec((4, blk, _EMB), lambda i: (0, i, 0))

    def full(a):
        return pl.BlockSpec(a.shape, lambda i: (0,) * a.ndim)

    out = pl.pallas_call(
        _tc_body,
        grid=(nb,),
        in_specs=[row_spec, row_spec, row_spec, row_spec, neg_spec, neg_spec,
                  full(w1t), full(w1b), full(w2), full(wp1), full(wp2),
                  full(b1), full(b2), full(bp)],
        out_specs=[pl.BlockSpec((blk,), lambda i: (i,)),
                   pl.BlockSpec((4, blk), lambda i: (0, i))],
        out_shape=[jax.ShapeDtypeStruct((B,), jnp.float32),
                   jax.ShapeDtypeStruct((4, B), jnp.float32)],
    )(gu, gv, ue, ve, gn.reshape(4, B, _EMB), ne.reshape(4, B, _EMB),
      w1t, w1b, w2, wp1, wp2, b1, b2, bp)
    return out


def kernel(u, v, n, gmf_u_emb, gmf_v_emb, u_emb, v_emb, W1, b1, W2, b2, Wp, bp):
    B = u.shape[0]
    nneg = n.shape[1]
    BN = B * nneg
    cb = B // _NW
    cn = BN // _NW
    u2 = u.astype(jnp.int32).reshape(_NW, cb // _CHUNK, _CHUNK)
    v2 = v.astype(jnp.int32).reshape(_NW, cb // _CHUNK, _CHUNK)
    n2 = n.astype(jnp.int32).T.reshape(_NW, cn // _CHUNK, _CHUNK)

    gmf_u_r, gmf_v_r, u_r, v_r = _tc_transpose4(
        gmf_u_emb.T, gmf_v_emb.T, u_emb.T, v_emb.T)

    gu, gv, ue, ve, gn, ne = _sc_gather(
        u2, v2, n2, gmf_u_r, gmf_v_r, u_r, v_r, B, BN)

    w1t, w1b = W1[:_EMB], W1[_EMB:]
    wp1 = Wp[:_EMB, 0].reshape(1, _EMB)
    wp2 = Wp[_EMB:, 0].reshape(1, 16)
    po, pn = _tc_dense(gu, gv, ue, ve, gn, ne, w1t, w1b, W2,
                       wp1, wp2, b1.reshape(1, _EMB), b2.reshape(1, 16),
                       bp.reshape(1, 1))
    pred = po
    pred_n = pn.T.reshape(-1)
    return (pred, pred_n)


# speedup vs baseline: 1.4791x; 1.4791x over previous
import functools

import jax
import jax.numpy as jnp
from jax import lax
from jax.experimental import pallas as pl
from jax.experimental.pallas import tpu as pltpu
from jax.experimental.pallas import tpu_sc as plsc

_NC, _NS = 2, 16
_NW = _NC * _NS
_EMB = 32
_CHUNK = 128


def _sc_gather(u2, v2, n2, gmf_u, gmf_v, u_t, v_t, B, BN):
    cb = B // _NW
    cn = BN // _NW
    kb = cb // _CHUNK
    kn = cn // _CHUNK
    mesh = plsc.VectorSubcoreMesh(core_axis_name="c", subcore_axis_name="s",
                                  num_cores=_NC, num_subcores=_NS)
    out_type = tuple(
        jax.ShapeDtypeStruct((sz, _EMB), jnp.float32)
        for sz in (B, B, B, B, BN, BN)
    )
    scratch = [
        pltpu.VMEM((kb, _CHUNK), jnp.int32),
        pltpu.VMEM((kb, _CHUNK), jnp.int32),
        pltpu.VMEM((kn, _CHUNK), jnp.int32),
        pltpu.VMEM((cn, _EMB), jnp.float32),
        pltpu.SemaphoreType.DMA,
    ]

    @functools.partial(pl.kernel, mesh=mesh, out_type=out_type,
                       scratch_types=scratch,
                       compiler_params=pltpu.CompilerParams(
                           use_tc_tiling_on_sc=False))
    def k(u_h, v_h, n_h, gmfu_h, gmfv_h, ut_h, vt_h,
          gu_o, gv_o, ue_o, ve_o, gn_o, ne_o,
          ui_v, vi_v, ni_v, rows_v, sem):
        w = lax.axis_index("s") * _NC + lax.axis_index("c")
        pltpu.sync_copy(u_h.at[w], ui_v)
        pltpu.sync_copy(v_h.at[w], vi_v)
        pltpu.sync_copy(n_h.at[w], ni_v)

        def do_set(tbl, idx2, nchunks, out, base, cnt):
            descs = []
            for i in range(nchunks):
                descs.append(pltpu.async_copy(
                    tbl.at[idx2.at[i]],
                    rows_v.at[pl.ds(i * _CHUNK, _CHUNK)], sem))
            for d in descs:
                d.wait()
            pltpu.sync_copy(rows_v.at[pl.ds(0, cnt)], out.at[pl.ds(base, cnt)])

        bu = w * cb
        bn = w * cn
        do_set(gmfu_h, ui_v, kb, gu_o, bu, cb)
        do_set(gmfv_h, vi_v, kb, gv_o, bu, cb)
        do_set(ut_h, ui_v, kb, ue_o, bu, cb)
        do_set(vt_h, vi_v, kb, ve_o, bu, cb)
        do_set(gmfv_h, ni_v, kn, gn_o, bn, cn)
        do_set(vt_h, ni_v, kn, ne_o, bn, cn)

    return k(u2, v2, n2, gmf_u, gmf_v, u_t, v_t)


def _transpose_body(a_r, b_r, c_r, d_r, ao_r, bo_r, co_r, do_r):
    blk = a_r.shape[1]
    for src, dst in ((a_r, ao_r), (b_r, bo_r), (c_r, co_r), (d_r, do_r)):
        t = src[...].T.reshape(blk // 4, 4, _EMB)
        dst[...] = jnp.concatenate([t[:, c, :] for c in range(4)], axis=1)


def _tc_transpose4(a, b, c, d):
    n = a.shape[1]
    blk = 2048
    grid = pl.cdiv(n, blk)
    in_spec = pl.BlockSpec((_EMB, blk), lambda i: (0, i))
    out_spec = pl.BlockSpec((blk // 4, 4 * _EMB), lambda i: (i, 0))
    outs = pl.pallas_call(
        _transpose_body,
        grid=(grid,),
        in_specs=[in_spec] * 4,
        out_specs=[out_spec] * 4,
        out_shape=[jax.ShapeDtypeStruct((n // 4, 4 * _EMB), jnp.float32)] * 4,
    )(a, b, c, d)
    return tuple(o.reshape(n, _EMB) for o in outs)


def _tc_body(gu_r, gv_r, ue_r, ve_r, gn_r, ne_r,
             w1t_r, w1b_r, w2_r, wp1_r, wp2_r, b1_r, b2_r, bp_r,
             po_r, pn_r):
    w1b = w1b_r[...]
    w2 = w2_r[...]
    wp1 = wp1_r[...]
    wp2 = wp2_r[...]
    b1 = b1_r[...]
    b2 = b2_r[...]
    bp = bp_r[0, 0]
    gu = gu_r[...]
    au = jnp.dot(ue_r[...], w1t_r[...], preferred_element_type=jnp.float32)

    def head(a_u, other, gmf_prod):
        h1 = jnp.maximum(
            a_u + jnp.dot(other, w1b, preferred_element_type=jnp.float32) + b1,
            0.0)
        h2 = jnp.maximum(
            jnp.dot(h1, w2, preferred_element_type=jnp.float32) + b2, 0.0)
        return (jnp.sum(gmf_prod * wp1, axis=1)
                + jnp.sum(h2 * wp2, axis=1) + bp)

    po_r[:] = head(au, ve_r[...], gu * gv_r[...])
    for j in range(4):
        pn_r[j, :] = head(au, ne_r[j], gu * gn_r[j])


def _tc_dense(gu, gv, ue, ve, gn, ne, w1t, w1b, w2, wp1, wp2, b1, b2, bp):
    B = gu.shape[0]
    blk = 2048
    nb = B // blk
    row_spec = pl.BlockSpec((blk, _EMB), lambda i: (i, 0))
    neg_spec = pl.BlockSpec((4, blk, _EMB), lambda i: (0, i, 0))

    def full(a):
        return pl.BlockSpec(a.shape, lambda i: (0,) * a.ndim)

    out = pl.pallas_call(
        _tc_body,
        grid=(nb,),
        in_specs=[row_spec, row_spec, row_spec, row_spec, neg_spec, neg_spec,
                  full(w1t), full(w1b), full(w2), full(wp1), full(wp2),
                  full(b1), full(b2), full(bp)],
        out_specs=[pl.BlockSpec((blk,), lambda i: (i,)),
                   pl.BlockSpec((4, blk), lambda i: (0, i))],
        out_shape=[jax.ShapeDtypeStruct((B,), jnp.float32),
                   jax.ShapeDtypeStruct((4, B), jnp.float32)],
    )(gu, gv, ue, ve, gn.reshape(4, B, _EMB), ne.reshape(4, B, _EMB),
      w1t, w1b, w2, wp1, wp2, b1, b2, bp)
    return out


def kernel(u, v, n, gmf_u_emb, gmf_v_emb, u_emb, v_emb, W1, b1, W2, b2, Wp, bp):
    B = u.shape[0]
    nneg = n.shape[1]
    BN = B * nneg
    cb = B // _NW
    cn = BN // _NW
    u2 = u.astype(jnp.int32).reshape(_NW, cb // _CHUNK, _CHUNK)
    v2 = v.astype(jnp.int32).reshape(_NW, cb // _CHUNK, _CHUNK)
    n2 = n.astype(jnp.int32).T.reshape(_NW, cn // _CHUNK, _CHUNK)

    gmf_u_r, gmf_v_r, u_r, v_r = _tc_transpose4(
        gmf_u_emb.T, gmf_v_emb.T, u_emb.T, v_emb.T)

    gu, gv, ue, ve, gn, ne = _sc_gather(
        u2, v2, n2, gmf_u_r, gmf_v_r, u_r, v_r, B, BN)

    w1t, w1b = W1[:_EMB], W1[_EMB:]
    wp1 = Wp[:_EMB, 0].reshape(1, _EMB)
    wp2 = Wp[_EMB:, 0].reshape(1, 16)
    po, pn = _tc_dense(gu, gv, ue, ve, gn, ne, w1t, w1b, W2,
                       wp1, wp2, b1.reshape(1, _EMB), b2.reshape(1, 16),
                       bp.reshape(1, 1))
    pred = po
    pred_n = pn.T.reshape(-1)
    return (pred, pred_n)


# speedup vs baseline: 3.6517x; 2.4688x over previous
import functools

import jax
import jax.numpy as jnp
from jax import lax
from jax.experimental import pallas as pl
from jax.experimental.pallas import tpu as pltpu
from jax.experimental.pallas import tpu_sc as plsc

_NC, _NS = 2, 16
_NW = _NC * _NS
_EMB = 32
_FW = 4 * _EMB
_CHUNK = 128
_GCH = 256


def _fuse_body(a_r, b_r, c_r, d_r, f_r):
    stacked = jnp.concatenate(
        [a_r[...], b_r[...], c_r[...], d_r[...]], axis=0)
    f_r[...] = stacked.T


def _tc_fuse_tables(a, b, c, d):
    n = a.shape[1]
    blk = 2048
    grid = pl.cdiv(n, blk)
    in_spec = pl.BlockSpec((_EMB, blk), lambda i: (0, i))
    return pl.pallas_call(
        _fuse_body,
        grid=(grid,),
        in_specs=[in_spec] * 4,
        out_specs=pl.BlockSpec((blk, _FW), lambda i: (i, 0)),
        out_shape=jax.ShapeDtypeStruct((n, _FW), jnp.float32),
    )(a, b, c, d)


def _sc_gather(u2, v2, n2, fused, B, BN):
    cb = B // _NW
    cn = BN // _NW
    mesh = plsc.VectorSubcoreMesh(core_axis_name="c", subcore_axis_name="s",
                                  num_cores=_NC, num_subcores=_NS)
    out_type = tuple(
        jax.ShapeDtypeStruct((sz, _FW), jnp.float32) for sz in (B, B, BN)
    )
    scratch = [
        pltpu.VMEM((cb // _CHUNK, _CHUNK), jnp.int32),
        pltpu.VMEM((cb // _CHUNK, _CHUNK), jnp.int32),
        pltpu.VMEM((cn // _CHUNK, _CHUNK), jnp.int32),
        pltpu.VMEM((_GCH, _FW), jnp.float32),
        pltpu.VMEM((_GCH, _FW), jnp.float32),
        pltpu.SemaphoreType.DMA,
        pltpu.SemaphoreType.DMA,
    ]

    @functools.partial(pl.kernel, mesh=mesh, out_type=out_type,
                       scratch_types=scratch,
                       compiler_params=pltpu.CompilerParams(
                           use_tc_tiling_on_sc=False))
    def k(u_h, v_h, n_h, f_h, ru_o, rv_o, rn_o,
          ui_v, vi_v, ni_v, buf_a, buf_b, sem_g, sem_w):
        w = lax.axis_index("s") * _NC + lax.axis_index("c")
        pltpu.sync_copy(u_h.at[w], ui_v)
        pltpu.sync_copy(v_h.at[w], vi_v)
        pltpu.sync_copy(n_h.at[w], ni_v)

        chunks = []
        for ci in range(cb // _GCH):
            chunks.append((ui_v, ci, ru_o, w * cb + ci * _GCH))
        for ci in range(cb // _GCH):
            chunks.append((vi_v, ci, rv_o, w * cb + ci * _GCH))
        for ci in range(cn // _GCH):
            chunks.append((ni_v, ci, rn_o, w * cn + ci * _GCH))

        k_per = _GCH // _CHUNK
        wb_prev = None
        for t, (idx_v, ci, out_h, base) in enumerate(chunks):
            buf = buf_a if t % 2 == 0 else buf_b
            descs = [
                pltpu.async_copy(
                    f_h.at[idx_v.at[ci * k_per + i]],
                    buf.at[pl.ds(i * _CHUNK, _CHUNK)], sem_g)
                for i in range(k_per)
            ]
            for d in descs:
                d.wait()
            if wb_prev is not None:
                wb_prev.wait()
            wb_prev = pltpu.async_copy(buf, out_h.at[pl.ds(base, _GCH)],
                                       sem_w)
        wb_prev.wait()

    return k(u2, v2, n2, fused)


def _tc_body(ru_r, rv_r, rn_r, w1t_r, w1b_r, w2_r, wp1_r, wp2_r,
             b1_r, b2_r, bp_r, po_r, pn_r):
    w1b = w1b_r[...]
    w2 = w2_r[...]
    wp1 = wp1_r[...]
    wp2 = wp2_r[...]
    b1 = b1_r[...]
    b2 = b2_r[...]
    bp = bp_r[0, 0]
    ru = ru_r[...]
    rv = rv_r[...]
    gu = ru[:, 0:_EMB]
    au = jnp.dot(ru[:, 2 * _EMB:3 * _EMB], w1t_r[...],
                 preferred_element_type=jnp.float32)

    def head(other_mlp, gmf_prod):
        h1 = jnp.maximum(
            au + jnp.dot(other_mlp, w1b, preferred_element_type=jnp.float32)
            + b1, 0.0)
        h2 = jnp.maximum(
            jnp.dot(h1, w2, preferred_element_type=jnp.float32) + b2, 0.0)
        return (jnp.sum(gmf_prod * wp1, axis=1)
                + jnp.sum(h2 * wp2, axis=1) + bp)

    po_r[:] = head(rv[:, 3 * _EMB:], gu * rv[:, _EMB:2 * _EMB])
    for j in range(4):
        rn = rn_r[j]
        pn_r[j, :] = head(rn[:, 3 * _EMB:], gu * rn[:, _EMB:2 * _EMB])


def _tc_dense(ru, rv, rn, w1t, w1b, w2, wp1, wp2, b1, b2, bp):
    B = ru.shape[0]
    blk = 2048
    nb = B // blk
    row_spec = pl.BlockSpec((blk, _FW), lambda i: (i, 0))
    neg_spec = pl.BlockSpec((4, blk, _FW), lambda i: (0, i, 0))

    def full(a):
        return pl.BlockSpec(a.shape, lambda i: (0,) * a.ndim)

    return pl.pallas_call(
        _tc_body,
        grid=(nb,),
        in_specs=[row_spec, row_spec, neg_spec,
                  full(w1t), full(w1b), full(w2), full(wp1), full(wp2),
                  full(b1), full(b2), full(bp)],
        out_specs=[pl.BlockSpec((blk,), lambda i: (i,)),
                   pl.BlockSpec((4, blk), lambda i: (0, i))],
        out_shape=[jax.ShapeDtypeStruct((B,), jnp.float32),
                   jax.ShapeDtypeStruct((4, B), jnp.float32)],
    )(ru, rv, rn.reshape(4, B, _FW), w1t, w1b, w2, wp1, wp2, b1, b2, bp)


def kernel(u, v, n, gmf_u_emb, gmf_v_emb, u_emb, v_emb, W1, b1, W2, b2, Wp, bp):
    B = u.shape[0]
    nneg = n.shape[1]
    BN = B * nneg
    cb = B // _NW
    cn = BN // _NW
    u2 = u.astype(jnp.int32).reshape(_NW, cb // _CHUNK, _CHUNK)
    v2 = v.astype(jnp.int32).reshape(_NW, cb // _CHUNK, _CHUNK)
    n2 = n.astype(jnp.int32).T.reshape(_NW, cn // _CHUNK, _CHUNK)

    fused = _tc_fuse_tables(gmf_u_emb.T, gmf_v_emb.T, u_emb.T, v_emb.T)
    ru, rv, rn = _sc_gather(u2, v2, n2, fused, B, BN)

    w1t, w1b = W1[:_EMB], W1[_EMB:]
    wp1 = Wp[:_EMB, 0].reshape(1, _EMB)
    wp2 = Wp[_EMB:, 0].reshape(1, 16)
    po, pn = _tc_dense(ru, rv, rn, w1t, w1b, W2, wp1, wp2,
                       b1.reshape(1, _EMB), b2.reshape(1, 16),
                       bp.reshape(1, 1))
    return (po, pn.T.reshape(-1))


# speedup vs baseline: 4.9421x; 1.3534x over previous
import functools

import jax
import jax.numpy as jnp
from jax import lax
from jax.experimental import pallas as pl
from jax.experimental.pallas import tpu as pltpu
from jax.experimental.pallas import tpu_sc as plsc

_NC, _NS = 2, 16
_NW = _NC * _NS
_EMB = 32
_FW = 4 * _EMB
_CHUNK = 128
_GCH = 256


def _fuse_body(a_r, b_r, c_r, d_r, f_r):
    stacked = jnp.concatenate(
        [a_r[...], b_r[...], c_r[...], d_r[...]], axis=0)
    f_r[...] = stacked.T


def _tc_fuse_tables(a, b, c, d):
    n = a.shape[1]
    blk = 4096
    grid = pl.cdiv(n, blk)
    in_spec = pl.BlockSpec((_EMB, blk), lambda i: (0, i))
    return pl.pallas_call(
        _fuse_body,
        grid=(grid,),
        in_specs=[in_spec] * 4,
        out_specs=pl.BlockSpec((blk, _FW), lambda i: (i, 0)),
        out_shape=jax.ShapeDtypeStruct((n, _FW), jnp.float32),
        compiler_params=pltpu.CompilerParams(vmem_limit_bytes=100 * 2**20),
    )(a, b, c, d)


def _sc_gather(u2, v2, n2, fused, B, BN):
    cb = B // _NW
    cn = BN // _NW
    mesh = plsc.VectorSubcoreMesh(core_axis_name="c", subcore_axis_name="s",
                                  num_cores=_NC, num_subcores=_NS)
    out_type = tuple(
        jax.ShapeDtypeStruct((sz, _FW), jnp.float32) for sz in (B, B, BN)
    )
    scratch = [
        pltpu.VMEM((cb // _CHUNK, _CHUNK), jnp.int32),
        pltpu.VMEM((cb // _CHUNK, _CHUNK), jnp.int32),
        pltpu.VMEM((cn // _CHUNK, _CHUNK), jnp.int32),
        pltpu.VMEM((_GCH, _FW), jnp.float32),
        pltpu.VMEM((_GCH, _FW), jnp.float32),
        pltpu.SemaphoreType.DMA,
        pltpu.SemaphoreType.DMA,
    ]

    @functools.partial(pl.kernel, mesh=mesh, out_type=out_type,
                       scratch_types=scratch,
                       compiler_params=pltpu.CompilerParams(
                           use_tc_tiling_on_sc=False))
    def k(u_h, v_h, n_h, f_h, ru_o, rv_o, rn_o,
          ui_v, vi_v, ni_v, buf_a, buf_b, sem_g, sem_w):
        w = lax.axis_index("s") * _NC + lax.axis_index("c")
        pltpu.sync_copy(u_h.at[w], ui_v)
        pltpu.sync_copy(v_h.at[w], vi_v)
        pltpu.sync_copy(n_h.at[w], ni_v)

        chunks = []
        for ci in range(cb // _GCH):
            chunks.append((ui_v, ci, ru_o, w * cb + ci * _GCH))
        for ci in range(cb // _GCH):
            chunks.append((vi_v, ci, rv_o, w * cb + ci * _GCH))
        for ci in range(cn // _GCH):
            chunks.append((ni_v, ci, rn_o, w * cn + ci * _GCH))

        k_per = _GCH // _CHUNK
        wb_prev = None
        for t, (idx_v, ci, out_h, base) in enumerate(chunks):
            buf = buf_a if t % 2 == 0 else buf_b
            descs = [
                pltpu.async_copy(
                    f_h.at[idx_v.at[ci * k_per + i]],
                    buf.at[pl.ds(i * _CHUNK, _CHUNK)], sem_g)
                for i in range(k_per)
            ]
            for d in descs:
                d.wait()
            if wb_prev is not None:
                wb_prev.wait()
            wb_prev = pltpu.async_copy(buf, out_h.at[pl.ds(base, _GCH)],
                                       sem_w)
        wb_prev.wait()

    return k(u2, v2, n2, fused)


def _tc_body(ru_r, rv_r, rn_r, w1u_r, w1v_r, w2p_r, wp1m_r, wones_r,
             wp2m_r, b1_r, b2p_r, bp_r, p_r):
    w1v = w1v_r[...]
    w2p = w2p_r[...]
    wp1m = wp1m_r[...]
    b1 = b1_r[...]
    b2p = b2p_r[...]
    ru = ru_r[...]
    au = jnp.dot(ru, w1u_r[...], preferred_element_type=jnp.float32)

    p = jnp.zeros(p_r.shape, jnp.float32) + bp_r[0, 0]
    heads = [rv_r[...]] + [rn_r[j] for j in range(4)]
    for h, rows in enumerate(heads):
        h1 = jnp.maximum(
            au + jnp.dot(rows, w1v, preferred_element_type=jnp.float32)
            + b1, 0.0)
        h2p = jnp.maximum(
            jnp.dot(h1, w2p, preferred_element_type=jnp.float32) + b2p, 0.0)
        gmf_w = ru * jnp.roll(rows, -_EMB, axis=1) * wp1m
        p = (p
             + jnp.dot(gmf_w, wones_r[h],
                       preferred_element_type=jnp.float32)
             + jnp.dot(h2p, wp2m_r[h], preferred_element_type=jnp.float32))
    p_r[...] = p


def _tc_dense(ru, rv, rn, w1u, w1v, w2p, wp1m, wones, wp2m, b1, b2p, bp):
    B = ru.shape[0]
    blk = 2048
    nb = B // blk
    row_spec = pl.BlockSpec((blk, _FW), lambda i: (i, 0))
    neg_spec = pl.BlockSpec((4, blk, _FW), lambda i: (0, i, 0))

    def full(a):
        return pl.BlockSpec(a.shape, lambda i: (0,) * a.ndim)

    return pl.pallas_call(
        _tc_body,
        grid=(nb,),
        in_specs=[row_spec, row_spec, neg_spec,
                  full(w1u), full(w1v), full(w2p), full(wp1m), full(wones),
                  full(wp2m), full(b1), full(b2p), full(bp)],
        out_specs=pl.BlockSpec((blk, 8), lambda i: (i, 0)),
        out_shape=jax.ShapeDtypeStruct((B, 8), jnp.float32),
    )(ru, rv, rn.reshape(4, B, _FW), w1u, w1v, w2p, wp1m, wones, wp2m,
      b1, b2p, bp)


def kernel(u, v, n, gmf_u_emb, gmf_v_emb, u_emb, v_emb, W1, b1, W2, b2, Wp, bp):
    B = u.shape[0]
    nneg = n.shape[1]
    BN = B * nneg
    cb = B // _NW
    cn = BN // _NW
    u2 = u.astype(jnp.int32).reshape(_NW, cb // _CHUNK, _CHUNK)
    v2 = v.astype(jnp.int32).reshape(_NW, cb // _CHUNK, _CHUNK)
    n2 = n.astype(jnp.int32).T.reshape(_NW, cn // _CHUNK, _CHUNK)

    fused = _tc_fuse_tables(gmf_u_emb.T, gmf_v_emb.T, u_emb.T, v_emb.T)
    ru, rv, rn = _sc_gather(u2, v2, n2, fused, B, BN)

    w1u = jnp.zeros((_FW, _EMB), jnp.float32).at[2 * _EMB:3 * _EMB].set(
        W1[:_EMB])
    w1v = jnp.zeros((_FW, _EMB), jnp.float32).at[3 * _EMB:].set(W1[_EMB:])
    w2p = jnp.zeros((_EMB, _FW), jnp.float32).at[:, :16].set(W2)
    wp1m = jnp.zeros((1, _FW), jnp.float32).at[0, :_EMB].set(Wp[:_EMB, 0])
    wones = jnp.zeros((5, _FW, 8), jnp.float32).at[
        jnp.arange(5)[:, None], jnp.arange(_EMB)[None, :],
        jnp.arange(5)[:, None]].set(1.0)
    wp2m = jnp.zeros((5, _FW, 8), jnp.float32).at[
        jnp.arange(5)[:, None], jnp.arange(16)[None, :],
        jnp.arange(5)[:, None]].set(Wp[_EMB:, 0][None, :])
    b2p = jnp.zeros((1, _FW), jnp.float32).at[0, :16].set(b2)
    p = _tc_dense(ru, rv, rn, w1u, w1v, w2p, wp1m, wones, wp2m,
                  b1.reshape(1, _EMB), b2p, bp.reshape(1, 1))
    return (p[:, 0], p[:, 1:5].reshape(-1))


# speedup vs baseline: 5.5807x; 1.1292x over previous
import functools

import jax
import jax.numpy as jnp
from jax import lax
from jax.experimental import pallas as pl
from jax.experimental.pallas import tpu as pltpu
from jax.experimental.pallas import tpu_sc as plsc

_NC, _NS = 2, 16
_NW = _NC * _NS
_EMB = 32
_FW = 4 * _EMB
_CHUNK = 128
_GCH = 256


def _fuse_body(a_r, b_r, c_r, d_r, f_r):
    stacked = jnp.concatenate(
        [a_r[...], b_r[...], c_r[...], d_r[...]], axis=0)
    f_r[...] = stacked.T


def _tc_fuse_tables(a, b, c, d):
    n = a.shape[1]
    blk = 8192
    grid = pl.cdiv(n, blk)
    in_spec = pl.BlockSpec((_EMB, blk), lambda i: (0, i))
    return pl.pallas_call(
        _fuse_body,
        grid=(grid,),
        in_specs=[in_spec] * 4,
        out_specs=pl.BlockSpec((blk, _FW), lambda i: (i, 0)),
        out_shape=jax.ShapeDtypeStruct((n, _FW), jnp.float32),
        compiler_params=pltpu.CompilerParams(vmem_limit_bytes=100 * 2**20),
    )(a, b, c, d)


def _sc_gather(u2, v2, n2, fused, B, BN):
    cb = B // _NW
    cn = BN // _NW
    mesh = plsc.VectorSubcoreMesh(core_axis_name="c", subcore_axis_name="s",
                                  num_cores=_NC, num_subcores=_NS)
    out_type = tuple(
        jax.ShapeDtypeStruct((sz, _FW), jnp.float32) for sz in (B, B, BN)
    )
    scratch = [
        pltpu.VMEM((cb // _CHUNK, _CHUNK), jnp.int32),
        pltpu.VMEM((cb // _CHUNK, _CHUNK), jnp.int32),
        pltpu.VMEM((cn // _CHUNK, _CHUNK), jnp.int32),
        pltpu.VMEM((_GCH, _FW), jnp.float32),
        pltpu.VMEM((_GCH, _FW), jnp.float32),
        pltpu.SemaphoreType.DMA,
        pltpu.SemaphoreType.DMA,
    ]

    @functools.partial(pl.kernel, mesh=mesh, out_type=out_type,
                       scratch_types=scratch,
                       compiler_params=pltpu.CompilerParams(
                           use_tc_tiling_on_sc=False))
    def k(u_h, v_h, n_h, f_h, ru_o, rv_o, rn_o,
          ui_v, vi_v, ni_v, buf_a, buf_b, sem_g, sem_w):
        w = lax.axis_index("s") * _NC + lax.axis_index("c")
        pltpu.sync_copy(u_h.at[w], ui_v)
        pltpu.sync_copy(v_h.at[w], vi_v)
        pltpu.sync_copy(n_h.at[w], ni_v)

        chunks = []
        for ci in range(cb // _GCH):
            chunks.append((ui_v, ci, ru_o, w * cb + ci * _GCH))
        for ci in range(cb // _GCH):
            chunks.append((vi_v, ci, rv_o, w * cb + ci * _GCH))
        for ci in range(cn // _GCH):
            chunks.append((ni_v, ci, rn_o, w * cn + ci * _GCH))

        k_per = _GCH // _CHUNK
        wb_prev = None
        for t, (idx_v, ci, out_h, base) in enumerate(chunks):
            buf = buf_a if t % 2 == 0 else buf_b
            descs = [
                pltpu.async_copy(
                    f_h.at[idx_v.at[ci * k_per + i]],
                    buf.at[pl.ds(i * _CHUNK, _CHUNK)], sem_g)
                for i in range(k_per)
            ]
            for d in descs:
                d.wait()
            if wb_prev is not None:
                wb_prev.wait()
            wb_prev = pltpu.async_copy(buf, out_h.at[pl.ds(base, _GCH)],
                                       sem_w)
        wb_prev.wait()

    return k(u2, v2, n2, fused)


def _tc_body(ru_r, rv_r, rn_r, w1u_r, w1v_r, w2p_r, wp1m_r, wones_r,
             wp2m_r, b1_r, b2p_r, bp_r, p_r):
    w1v = w1v_r[...]
    w2p = w2p_r[...]
    wp1m = wp1m_r[...]
    b1 = b1_r[...]
    b2p = b2p_r[...]
    ru = ru_r[...]
    au = jnp.dot(ru, w1u_r[...], preferred_element_type=jnp.float32)

    p = jnp.zeros(p_r.shape, jnp.float32) + bp_r[0, 0]
    heads = [rv_r[...]] + [rn_r[j] for j in range(4)]
    for h, rows in enumerate(heads):
        h1 = jnp.maximum(
            au + jnp.dot(rows, w1v, preferred_element_type=jnp.float32)
            + b1, 0.0)
        h2p = jnp.maximum(
            jnp.dot(h1, w2p, preferred_element_type=jnp.float32) + b2p, 0.0)
        gmf_w = ru * jnp.roll(rows, -_EMB, axis=1) * wp1m
        p = (p
             + jnp.dot(gmf_w, wones_r[h],
                       preferred_element_type=jnp.float32)
             + jnp.dot(h2p, wp2m_r[h], preferred_element_type=jnp.float32))
    p_r[...] = p


def _tc_dense(ru, rv, rn, w1u, w1v, w2p, wp1m, wones, wp2m, b1, b2p, bp):
    B = ru.shape[0]
    blk = 2048
    nb = B // blk
    row_spec = pl.BlockSpec((blk, _FW), lambda i: (i, 0))
    neg_spec = pl.BlockSpec((4, blk, _FW), lambda i: (0, i, 0))

    def full(a):
        return pl.BlockSpec(a.shape, lambda i: (0,) * a.ndim)

    return pl.pallas_call(
        _tc_body,
        grid=(nb,),
        in_specs=[row_spec, row_spec, neg_spec,
                  full(w1u), full(w1v), full(w2p), full(wp1m), full(wones),
                  full(wp2m), full(b1), full(b2p), full(bp)],
        out_specs=pl.BlockSpec((blk, 8), lambda i: (i, 0)),
        out_shape=jax.ShapeDtypeStruct((B, 8), jnp.float32),
    )(ru, rv, rn.reshape(4, B, _FW), w1u, w1v, w2p, wp1m, wones, wp2m,
      b1, b2p, bp)


def kernel(u, v, n, gmf_u_emb, gmf_v_emb, u_emb, v_emb, W1, b1, W2, b2, Wp, bp):
    B = u.shape[0]
    nneg = n.shape[1]
    BN = B * nneg
    cb = B // _NW
    cn = BN // _NW
    u2 = u.astype(jnp.int32).reshape(_NW, cb // _CHUNK, _CHUNK)
    v2 = v.astype(jnp.int32).reshape(_NW, cb // _CHUNK, _CHUNK)
    n2 = n.astype(jnp.int32).T.reshape(_NW, cn // _CHUNK, _CHUNK)

    fused = _tc_fuse_tables(gmf_u_emb.T, gmf_v_emb.T, u_emb.T, v_emb.T)
    ru, rv, rn = _sc_gather(u2, v2, n2, fused, B, BN)

    w1u = jnp.zeros((_FW, _EMB), jnp.float32).at[2 * _EMB:3 * _EMB].set(
        W1[:_EMB])
    w1v = jnp.zeros((_FW, _EMB), jnp.float32).at[3 * _EMB:].set(W1[_EMB:])
    w2p = jnp.zeros((_EMB, _FW), jnp.float32).at[:, :16].set(W2)
    wp1m = jnp.zeros((1, _FW), jnp.float32).at[0, :_EMB].set(Wp[:_EMB, 0])
    wones = jnp.zeros((5, _FW, 8), jnp.float32).at[
        jnp.arange(5)[:, None], jnp.arange(_EMB)[None, :],
        jnp.arange(5)[:, None]].set(1.0)
    wp2m = jnp.zeros((5, _FW, 8), jnp.float32).at[
        jnp.arange(5)[:, None], jnp.arange(16)[None, :],
        jnp.arange(5)[:, None]].set(Wp[_EMB:, 0][None, :])
    b2p = jnp.zeros((1, _FW), jnp.float32).at[0, :16].set(b2)
    p = _tc_dense(ru, rv, rn, w1u, w1v, w2p, wp1m, wones, wp2m,
                  b1.reshape(1, _EMB), b2p, bp.reshape(1, 1))
    return (p[:, 0], p[:, 1:5].reshape(-1))


# speedup vs baseline: 5.6887x; 1.0194x over previous
import functools

import jax
import jax.numpy as jnp
from jax import lax
from jax.experimental import pallas as pl
from jax.experimental.pallas import tpu as pltpu
from jax.experimental.pallas import tpu_sc as plsc

_NC, _NS = 2, 16
_NW = _NC * _NS
_EMB = 32
_FW = 4 * _EMB
_CHUNK = 128
_GCH = 256


def _fuse_body(a_r, b_r, c_r, d_r, f_r):
    stacked = jnp.concatenate(
        [a_r[...], b_r[...], c_r[...], d_r[...]], axis=0)
    f_r[...] = stacked.T


def _tc_fuse_tables(a, b, c, d):
    n = a.shape[1]
    blk = 16384
    grid = pl.cdiv(n, blk)
    in_spec = pl.BlockSpec((_EMB, blk), lambda i: (0, i))
    return pl.pallas_call(
        _fuse_body,
        grid=(grid,),
        in_specs=[in_spec] * 4,
        out_specs=pl.BlockSpec((blk, _FW), lambda i: (i, 0)),
        out_shape=jax.ShapeDtypeStruct((n, _FW), jnp.float32),
        compiler_params=pltpu.CompilerParams(vmem_limit_bytes=100 * 2**20),
    )(a, b, c, d)


def _sc_gather(u2, v2, n2, fused, B, BN):
    cb = B // _NW
    cn = BN // _NW
    mesh = plsc.VectorSubcoreMesh(core_axis_name="c", subcore_axis_name="s",
                                  num_cores=_NC, num_subcores=_NS)
    out_type = tuple(
        jax.ShapeDtypeStruct((sz, _FW), jnp.float32) for sz in (B, B, BN)
    )
    scratch = [
        pltpu.VMEM((cb // _CHUNK, _CHUNK), jnp.int32),
        pltpu.VMEM((cb // _CHUNK, _CHUNK), jnp.int32),
        pltpu.VMEM((cn // _CHUNK, _CHUNK), jnp.int32),
        pltpu.VMEM((_GCH, _FW), jnp.float32),
        pltpu.VMEM((_GCH, _FW), jnp.float32),
        pltpu.SemaphoreType.DMA,
        pltpu.SemaphoreType.DMA,
    ]

    @functools.partial(pl.kernel, mesh=mesh, out_type=out_type,
                       scratch_types=scratch,
                       compiler_params=pltpu.CompilerParams(
                           use_tc_tiling_on_sc=False))
    def k(u_h, v_h, n_h, f_h, ru_o, rv_o, rn_o,
          ui_v, vi_v, ni_v, buf_a, buf_b, sem_g, sem_w):
        w = lax.axis_index("s") * _NC + lax.axis_index("c")
        pltpu.sync_copy(u_h.at[w], ui_v)
        pltpu.sync_copy(v_h.at[w], vi_v)
        pltpu.sync_copy(n_h.at[w], ni_v)

        chunks = []
        for ci in range(cb // _GCH):
            chunks.append((ui_v, ci, ru_o, w * cb + ci * _GCH))
        for ci in range(cb // _GCH):
            chunks.append((vi_v, ci, rv_o, w * cb + ci * _GCH))
        for ci in range(cn // _GCH):
            chunks.append((ni_v, ci, rn_o, w * cn + ci * _GCH))

        k_per = _GCH // _CHUNK
        wb_prev = None
        for t, (idx_v, ci, out_h, base) in enumerate(chunks):
            buf = buf_a if t % 2 == 0 else buf_b
            descs = [
                pltpu.async_copy(
                    f_h.at[idx_v.at[ci * k_per + i]],
                    buf.at[pl.ds(i * _CHUNK, _CHUNK)], sem_g)
                for i in range(k_per)
            ]
            for d in descs:
                d.wait()
            if wb_prev is not None:
                wb_prev.wait()
            wb_prev = pltpu.async_copy(buf, out_h.at[pl.ds(base, _GCH)],
                                       sem_w)
        wb_prev.wait()

    return k(u2, v2, n2, fused)


def _tc_body(ru_r, rv_r, rn_r, w1u_r, w1v_r, w2p_r, wp1m_r, wones_r,
             wp2m_r, b1_r, b2p_r, bp_r, p_r):
    w1v = w1v_r[...]
    w2p = w2p_r[...]
    wp1m = wp1m_r[...]
    b1 = b1_r[...]
    b2p = b2p_r[...]
    ru = ru_r[...]
    au = jnp.dot(ru, w1u_r[...], preferred_element_type=jnp.float32)

    p = jnp.zeros(p_r.shape, jnp.float32) + bp_r[0, 0]
    heads = [rv_r[...]] + [rn_r[j] for j in range(4)]
    for h, rows in enumerate(heads):
        h1 = jnp.maximum(
            au + jnp.dot(rows, w1v, preferred_element_type=jnp.float32)
            + b1, 0.0)
        h2p = jnp.maximum(
            jnp.dot(h1, w2p, preferred_element_type=jnp.float32) + b2p, 0.0)
        gmf_w = ru * jnp.roll(rows, -_EMB, axis=1) * wp1m
        p = (p
             + jnp.dot(gmf_w, wones_r[h],
                       preferred_element_type=jnp.float32)
             + jnp.dot(h2p, wp2m_r[h], preferred_element_type=jnp.float32))
    p_r[...] = p


def _tc_dense(ru, rv, rn, w1u, w1v, w2p, wp1m, wones, wp2m, b1, b2p, bp):
    B = ru.shape[0]
    blk = 2048
    nb = B // blk
    row_spec = pl.BlockSpec((blk, _FW), lambda i: (i, 0))
    neg_spec = pl.BlockSpec((4, blk, _FW), lambda i: (0, i, 0))

    def full(a):
        return pl.BlockSpec(a.shape, lambda i: (0,) * a.ndim)

    return pl.pallas_call(
        _tc_body,
        grid=(nb,),
        in_specs=[row_spec, row_spec, neg_spec,
                  full(w1u), full(w1v), full(w2p), full(wp1m), full(wones),
                  full(wp2m), full(b1), full(b2p), full(bp)],
        out_specs=pl.BlockSpec((blk, 8), lambda i: (i, 0)),
        out_shape=jax.ShapeDtypeStruct((B, 8), jnp.float32),
    )(ru, rv, rn.reshape(4, B, _FW), w1u, w1v, w2p, wp1m, wones, wp2m,
      b1, b2p, bp)


def kernel(u, v, n, gmf_u_emb, gmf_v_emb, u_emb, v_emb, W1, b1, W2, b2, Wp, bp):
    B = u.shape[0]
    nneg = n.shape[1]
    BN = B * nneg
    cb = B // _NW
    cn = BN // _NW
    u2 = u.astype(jnp.int32).reshape(_NW, cb // _CHUNK, _CHUNK)
    v2 = v.astype(jnp.int32).reshape(_NW, cb // _CHUNK, _CHUNK)
    n2 = n.astype(jnp.int32).T.reshape(_NW, cn // _CHUNK, _CHUNK)

    fused = _tc_fuse_tables(gmf_u_emb.T, gmf_v_emb.T, u_emb.T, v_emb.T)
    ru, rv, rn = _sc_gather(u2, v2, n2, fused, B, BN)

    w1u = jnp.zeros((_FW, _EMB), jnp.float32).at[2 * _EMB:3 * _EMB].set(
        W1[:_EMB])
    w1v = jnp.zeros((_FW, _EMB), jnp.float32).at[3 * _EMB:].set(W1[_EMB:])
    w2p = jnp.zeros((_EMB, _FW), jnp.float32).at[:, :16].set(W2)
    wp1m = jnp.zeros((1, _FW), jnp.float32).at[0, :_EMB].set(Wp[:_EMB, 0])
    wones = jnp.zeros((5, _FW, 8), jnp.float32).at[
        jnp.arange(5)[:, None], jnp.arange(_EMB)[None, :],
        jnp.arange(5)[:, None]].set(1.0)
    wp2m = jnp.zeros((5, _FW, 8), jnp.float32).at[
        jnp.arange(5)[:, None], jnp.arange(16)[None, :],
        jnp.arange(5)[:, None]].set(Wp[_EMB:, 0][None, :])
    b2p = jnp.zeros((1, _FW), jnp.float32).at[0, :16].set(b2)
    p = _tc_dense(ru, rv, rn, w1u, w1v, w2p, wp1m, wones, wp2m,
                  b1.reshape(1, _EMB), b2p, bp.reshape(1, 1))
    return (p[:, 0], p[:, 1:5].reshape(-1))
